# baseline (speedup 1.0000x reference)
import jax
import jax.numpy as jnp
from jax.experimental import pallas as pl
from jax.experimental.pallas import tpu as pltpu

N = 4096
D = 512
H = 256
L = 2

BI = 2048
BK = 512
NI = N // BI
NK = N // BK


def _proj_kernel(h_ref, w_ref, out_ref):
    out_ref[...] = jnp.dot(h_ref[...].astype(jnp.bfloat16),
                           w_ref[...].astype(jnp.bfloat16),
                           preferred_element_type=jnp.float32
                           ).astype(jnp.bfloat16)


def _agg_kernel(bw0_ref, bw1_ref, fw0_ref, fw1_ref, xw_ref, wl_ref,
                bpre_ref, blin_ref, h_ref, out_ref, acc_ref):
    k = pl.program_id(1)

    @pl.when(k == 0)
    def _init():
        acc_ref[...] = jnp.zeros_like(acc_ref)

    xw = xw_ref[...]
    bw0 = bw0_ref[...].astype(jnp.bfloat16)
    bw1 = bw1_ref[...].astype(jnp.bfloat16)
    fw0 = fw0_ref[...].astype(jnp.bfloat16)
    fw1 = fw1_ref[...].astype(jnp.bfloat16)
    acc_ref[:, :H] += (
        jnp.dot(bw0, xw[:, 0:H], preferred_element_type=jnp.float32)
        + jnp.dot(bw1, xw[:, H:2 * H], preferred_element_type=jnp.float32))
    acc_ref[:, H:] += (
        jnp.dot(fw0, xw[:, 2 * H:3 * H], preferred_element_type=jnp.float32)
        + jnp.dot(fw1, xw[:, 3 * H:4 * H], preferred_element_type=jnp.float32))

    @pl.when(k == NK - 1)
    def _finalize():
        s = jnp.maximum(acc_ref[...] + bpre_ref[...], 0.0).astype(jnp.bfloat16)
        out_ref[...] = (jnp.dot(s, wl_ref[...].astype(jnp.bfloat16),
                                preferred_element_type=jnp.float32)
                        + blin_ref[...] + h_ref[...])


def _bigcn_layer(h, bw0, bw1, fw0, fw1, Wcat, wl, bpre, blin):
    xw = pl.pallas_call(
        _proj_kernel,
        grid=(NI,),
        in_specs=[pl.BlockSpec((BI, D), lambda i: (i, 0)),
                  pl.BlockSpec((D, 4 * H), lambda i: (0, 0))],
        out_specs=pl.BlockSpec((BI, 4 * H), lambda i: (i, 0)),
        out_shape=jax.ShapeDtypeStruct((N, 4 * H), jnp.bfloat16),
        compiler_params=pltpu.CompilerParams(
            dimension_semantics=("arbitrary",)),
    )(h, Wcat)

    adj_spec = pl.BlockSpec((BI, BK), lambda i, k: (i, k))
    out = pl.pallas_call(
        _agg_kernel,
        grid=(NI, NK),
        in_specs=[adj_spec, adj_spec, adj_spec, adj_spec,
                  pl.BlockSpec((BK, 4 * H), lambda i, k: (k, 0)),
                  pl.BlockSpec((D, D), lambda i, k: (0, 0)),
                  pl.BlockSpec((1, D), lambda i, k: (0, 0)),
                  pl.BlockSpec((1, D), lambda i, k: (0, 0)),
                  pl.BlockSpec((BI, D), lambda i, k: (i, 0))],
        out_specs=pl.BlockSpec((BI, D), lambda i, k: (i, 0)),
        out_shape=jax.ShapeDtypeStruct((N, D), jnp.float32),
        scratch_shapes=[pltpu.VMEM((BI, D), jnp.float32)],
        compiler_params=pltpu.CompilerParams(
            dimension_semantics=("parallel", "arbitrary"),
            vmem_limit_bytes=100 * 1024 * 1024),
    )(bw0, bw1, fw0, fw1, xw, wl, bpre, blin, h)
    return out


def kernel(embs, fw_adj_0, fw_adj_1, bw_adj_0, bw_adj_1,
           W_fw, b_fw, W_bw, b_bw, W_lin, b_lin):
    h = embs
    for l in range(L):
        Wcat = jnp.concatenate(
            [W_bw[l, 0], W_bw[l, 1], W_fw[l, 0], W_fw[l, 1]], axis=1)
        bpre = jnp.concatenate(
            [b_bw[l, 0] + b_bw[l, 1], b_fw[l, 0] + b_fw[l, 1]])[None, :]
        blin = b_lin[l][None, :]
        h = _bigcn_layer(h, bw_adj_0, bw_adj_1, fw_adj_0, fw_adj_1,
                         Wcat, W_lin[l], bpre, blin)
    return h


# speedup vs baseline: 1.0298x; 1.0298x over previous
import jax
import jax.numpy as jnp
from jax.experimental import pallas as pl
from jax.experimental.pallas import tpu as pltpu

N = 4096
D = 512
H = 256
L = 2

BI = 2048
BK = 256
NI = N // BI
NK = N // BK


def _proj_kernel(h_ref, w_ref, out_ref):
    out_ref[...] = jnp.dot(h_ref[...].astype(jnp.bfloat16),
                           w_ref[...].astype(jnp.bfloat16),
                           preferred_element_type=jnp.float32
                           ).astype(jnp.bfloat16)


def _accumulate(bw0_ref, bw1_ref, fw0_ref, fw1_ref, xw_ref, acc_ref, k):
    @pl.when(k == 0)
    def _init():
        acc_ref[...] = jnp.zeros_like(acc_ref)

    xw = xw_ref[...]
    bw0 = bw0_ref[...].astype(jnp.bfloat16)
    bw1 = bw1_ref[...].astype(jnp.bfloat16)
    fw0 = fw0_ref[...].astype(jnp.bfloat16)
    fw1 = fw1_ref[...].astype(jnp.bfloat16)
    acc_ref[:, :H] += (
        jnp.dot(bw0, xw[:, 0:H], preferred_element_type=jnp.float32)
        + jnp.dot(bw1, xw[:, H:2 * H], preferred_element_type=jnp.float32))
    acc_ref[:, H:] += (
        jnp.dot(fw0, xw[:, 2 * H:3 * H], preferred_element_type=jnp.float32)
        + jnp.dot(fw1, xw[:, 3 * H:4 * H], preferred_element_type=jnp.float32))


def _finalize(acc_ref, wl_ref, bpre_ref, blin_ref, h_ref):
    s = jnp.maximum(acc_ref[...] + bpre_ref[...], 0.0).astype(jnp.bfloat16)
    return (jnp.dot(s, wl_ref[...].astype(jnp.bfloat16),
                    preferred_element_type=jnp.float32)
            + blin_ref[...] + h_ref[...])


def _agg_mid_kernel(bw0_ref, bw1_ref, fw0_ref, fw1_ref, xw_ref, wl_ref,
                    bpre_ref, blin_ref, h_ref, wnext_ref,
                    out_ref, xwn_ref, acc_ref):
    k = pl.program_id(1)
    _accumulate(bw0_ref, bw1_ref, fw0_ref, fw1_ref, xw_ref, acc_ref, k)

    @pl.when(k == NK - 1)
    def _epilogue():
        out = _finalize(acc_ref, wl_ref, bpre_ref, blin_ref, h_ref)
        out_ref[...] = out
        xwn_ref[...] = jnp.dot(out.astype(jnp.bfloat16),
                               wnext_ref[...].astype(jnp.bfloat16),
                               preferred_element_type=jnp.float32
                               ).astype(jnp.bfloat16)


def _agg_last_kernel(bw0_ref, bw1_ref, fw0_ref, fw1_ref, xw_ref, wl_ref,
                     bpre_ref, blin_ref, h_ref, out_ref, acc_ref):
    k = pl.program_id(1)
    _accumulate(bw0_ref, bw1_ref, fw0_ref, fw1_ref, xw_ref, acc_ref, k)

    @pl.when(k == NK - 1)
    def _epilogue():
        out_ref[...] = _finalize(acc_ref, wl_ref, bpre_ref, blin_ref, h_ref)


_adj_spec = pl.BlockSpec((BI, BK), lambda i, k: (i, k))
_common_in_specs = [
    _adj_spec, _adj_spec, _adj_spec, _adj_spec,
    pl.BlockSpec((BK, 4 * H), lambda i, k: (k, 0)),
    pl.BlockSpec((D, D), lambda i, k: (0, 0)),
    pl.BlockSpec((1, D), lambda i, k: (0, 0)),
    pl.BlockSpec((1, D), lambda i, k: (0, 0)),
    pl.BlockSpec((BI, D), lambda i, k: (i, 0)),
]
_out_spec = pl.BlockSpec((BI, D), lambda i, k: (i, 0))
_xw_spec = pl.BlockSpec((BI, 4 * H), lambda i, k: (i, 0))
_params = pltpu.CompilerParams(
    dimension_semantics=("parallel", "arbitrary"),
    vmem_limit_bytes=100 * 1024 * 1024)


def kernel(embs, fw_adj_0, fw_adj_1, bw_adj_0, bw_adj_1,
           W_fw, b_fw, W_bw, b_bw, W_lin, b_lin):
    Wcat = [jnp.concatenate(
        [W_bw[l, 0], W_bw[l, 1], W_fw[l, 0], W_fw[l, 1]], axis=1)
        for l in range(L)]
    bpre = [jnp.concatenate(
        [b_bw[l, 0] + b_bw[l, 1], b_fw[l, 0] + b_fw[l, 1]])[None, :]
        for l in range(L)]
    blin = [b_lin[l][None, :] for l in range(L)]

    xw1 = pl.pallas_call(
        _proj_kernel,
        grid=(NI,),
        in_specs=[pl.BlockSpec((BI, D), lambda i: (i, 0)),
                  pl.BlockSpec((D, 4 * H), lambda i: (0, 0))],
        out_specs=pl.BlockSpec((BI, 4 * H), lambda i: (i, 0)),
        out_shape=jax.ShapeDtypeStruct((N, 4 * H), jnp.bfloat16),
        compiler_params=pltpu.CompilerParams(
            dimension_semantics=("arbitrary",)),
    )(embs, Wcat[0])

    h1, xw2 = pl.pallas_call(
        _agg_mid_kernel,
        grid=(NI, NK),
        in_specs=_common_in_specs + [
            pl.BlockSpec((D, 4 * H), lambda i, k: (0, 0))],
        out_specs=[_out_spec, _xw_spec],
        out_shape=[jax.ShapeDtypeStruct((N, D), jnp.float32),
                   jax.ShapeDtypeStruct((N, 4 * H), jnp.bfloat16)],
        scratch_shapes=[pltpu.VMEM((BI, D), jnp.float32)],
        compiler_params=_params,
    )(bw_adj_0, bw_adj_1, fw_adj_0, fw_adj_1, xw1, W_lin[0],
      bpre[0], blin[0], embs, Wcat[1])

    h2 = pl.pallas_call(
        _agg_last_kernel,
        grid=(NI, NK),
        in_specs=_common_in_specs,
        out_specs=_out_spec,
        out_shape=jax.ShapeDtypeStruct((N, D), jnp.float32),
        scratch_shapes=[pltpu.VMEM((BI, D), jnp.float32)],
        compiler_params=_params,
    )(bw_adj_0, bw_adj_1, fw_adj_0, fw_adj_1, xw2, W_lin[1],
      bpre[1], blin[1], h1)
    return h2


# speedup vs baseline: 1.1082x; 1.0762x over previous
import jax
import jax.numpy as jnp
from jax.experimental import pallas as pl
from jax.experimental.pallas import tpu as pltpu

N = 4096
D = 512
H = 256
L = 2

BI = 2048
BK = 256
NI = N // BI
NK = N // BK


def _proj_kernel(h_ref, w_ref, out_ref):
    out_ref[...] = jnp.dot(h_ref[...].astype(jnp.bfloat16),
                           w_ref[...].astype(jnp.bfloat16),
                           preferred_element_type=jnp.float32
                           ).astype(jnp.bfloat16)


def _accum(acc_ref, bw0_ref, bw1_ref, fw0_ref, fw1_ref, xw):
    bw0 = bw0_ref[...].astype(jnp.bfloat16)
    bw1 = bw1_ref[...].astype(jnp.bfloat16)
    fw0 = fw0_ref[...].astype(jnp.bfloat16)
    fw1 = fw1_ref[...].astype(jnp.bfloat16)
    acc_ref[:, :H] += (
        jnp.dot(bw0, xw[:, 0:H], preferred_element_type=jnp.float32)
        + jnp.dot(bw1, xw[:, H:2 * H], preferred_element_type=jnp.float32))
    acc_ref[:, H:] += (
        jnp.dot(fw0, xw[:, 2 * H:3 * H], preferred_element_type=jnp.float32)
        + jnp.dot(fw1, xw[:, 3 * H:4 * H], preferred_element_type=jnp.float32))


def _mega_kernel(bw0_ref, bw1_ref, fw0_ref, fw1_ref, xw1_ref, wl_ref,
                 bpre_ref, blin_ref, embs_ref, wcat2_ref,
                 out_ref, acc_ref, h1_ref, xw2_ref):
    l = pl.program_id(0)
    i = pl.program_id(1)
    k = pl.program_id(2)

    @pl.when(k == 0)
    def _init():
        acc_ref[...] = jnp.zeros_like(acc_ref)

    @pl.when(l == 0)
    def _accum_l0():
        _accum(acc_ref, bw0_ref, bw1_ref, fw0_ref, fw1_ref, xw1_ref[...])

    @pl.when(l == 1)
    def _accum_l1():
        _accum(acc_ref, bw0_ref, bw1_ref, fw0_ref, fw1_ref,
               xw2_ref[pl.ds(k * BK, BK), :])

    @pl.when(k == NK - 1)
    def _epilogue():
        s = jnp.maximum(acc_ref[...] + bpre_ref[0], 0.0).astype(jnp.bfloat16)
        lin = (jnp.dot(s, wl_ref[0].astype(jnp.bfloat16),
                       preferred_element_type=jnp.float32) + blin_ref[0])

        @pl.when(l == 0)
        def _emit_l1():
            h1 = lin + embs_ref[...]
            h1_ref[pl.ds(i * BI, BI), :] = h1
            xw2_ref[pl.ds(i * BI, BI), :] = jnp.dot(
                h1.astype(jnp.bfloat16), wcat2_ref[...],
                preferred_element_type=jnp.float32).astype(jnp.bfloat16)

        @pl.when(l == 1)
        def _emit_out():
            out_ref[...] = lin + h1_ref[pl.ds(i * BI, BI), :]


def kernel(embs, fw_adj_0, fw_adj_1, bw_adj_0, bw_adj_1,
           W_fw, b_fw, W_bw, b_bw, W_lin, b_lin):
    Wcat = [jnp.concatenate(
        [W_bw[l, 0], W_bw[l, 1], W_fw[l, 0], W_fw[l, 1]], axis=1)
        for l in range(L)]
    bpre = jnp.stack([
        jnp.concatenate([b_bw[l, 0] + b_bw[l, 1], b_fw[l, 0] + b_fw[l, 1]])
        for l in range(L)])[:, None, :]
    blin = b_lin[:, None, :]
    wcat2_bf16 = Wcat[1].astype(jnp.bfloat16)

    xw1 = pl.pallas_call(
        _proj_kernel,
        grid=(NI,),
        in_specs=[pl.BlockSpec((BI, D), lambda i: (i, 0)),
                  pl.BlockSpec((D, 4 * H), lambda i: (0, 0))],
        out_specs=pl.BlockSpec((BI, 4 * H), lambda i: (i, 0)),
        out_shape=jax.ShapeDtypeStruct((N, 4 * H), jnp.bfloat16),
        compiler_params=pltpu.CompilerParams(
            dimension_semantics=("arbitrary",)),
    )(embs, Wcat[0])

    adj_spec = pl.BlockSpec((BI, BK), lambda l, i, k: (i, k))
    out = pl.pallas_call(
        _mega_kernel,
        grid=(L, NI, NK),
        in_specs=[
            adj_spec, adj_spec, adj_spec, adj_spec,
            pl.BlockSpec((BK, 4 * H),
                         lambda l, i, k: (jnp.where(l == 0, k, 0), 0)),
            pl.BlockSpec((1, D, D), lambda l, i, k: (l, 0, 0)),
            pl.BlockSpec((1, 1, D), lambda l, i, k: (l, 0, 0)),
            pl.BlockSpec((1, 1, D), lambda l, i, k: (l, 0, 0)),
            pl.BlockSpec((BI, D),
                         lambda l, i, k: (jnp.where(l == 0, i, 0), 0)),
            pl.BlockSpec((D, 4 * H), lambda l, i, k: (0, 0)),
        ],
        out_specs=pl.BlockSpec((BI, D),
                               lambda l, i, k: (jnp.where(l == 1, i, 0), 0)),
        out_shape=jax.ShapeDtypeStruct((N, D), jnp.float32),
        scratch_shapes=[pltpu.VMEM((BI, D), jnp.float32),
                        pltpu.VMEM((N, D), jnp.float32),
                        pltpu.VMEM((N, 4 * H), jnp.bfloat16)],
        compiler_params=pltpu.CompilerParams(
            dimension_semantics=("arbitrary", "arbitrary", "arbitrary"),
            vmem_limit_bytes=100 * 1024 * 1024),
    )(bw_adj_0, bw_adj_1, fw_adj_0, fw_adj_1, xw1, W_lin,
      bpre, blin, embs, wcat2_bf16)
    return out


# speedup vs baseline: 1.1444x; 1.0326x over previous
import jax
import jax.numpy as jnp
from jax.experimental import pallas as pl
from jax.experimental.pallas import tpu as pltpu

N = 4096
D = 512
H = 256
L = 2

BI = 2048
BK = 256
NI = N // BI
NK = N // BK


def _accum(acc_ref, bw0_ref, bw1_ref, fw0_ref, fw1_ref, xw):
    bw0 = bw0_ref[...].astype(jnp.bfloat16)
    bw1 = bw1_ref[...].astype(jnp.bfloat16)
    fw0 = fw0_ref[...].astype(jnp.bfloat16)
    fw1 = fw1_ref[...].astype(jnp.bfloat16)
    acc_ref[:, :H] += (
        jnp.dot(bw0, xw[:, 0:H], preferred_element_type=jnp.float32)
        + jnp.dot(bw1, xw[:, H:2 * H], preferred_element_type=jnp.float32))
    acc_ref[:, H:] += (
        jnp.dot(fw0, xw[:, 2 * H:3 * H], preferred_element_type=jnp.float32)
        + jnp.dot(fw1, xw[:, 3 * H:4 * H], preferred_element_type=jnp.float32))


def _mega_kernel(bw0_ref, bw1_ref, fw0_ref, fw1_ref, embsk_ref, wl_ref,
                 bpre_ref, blin_ref, embs_ref, wcat1_ref, wcat2_ref,
                 out_ref, acc_ref, h1_ref, xw2_ref):
    l = pl.program_id(0)
    i = pl.program_id(1)
    k = pl.program_id(2)

    @pl.when(k == 0)
    def _init():
        acc_ref[...] = jnp.zeros_like(acc_ref)

    @pl.when(l == 0)
    def _accum_l0():
        xw1 = jnp.dot(embsk_ref[...].astype(jnp.bfloat16), wcat1_ref[...],
                      preferred_element_type=jnp.float32).astype(jnp.bfloat16)
        _accum(acc_ref, bw0_ref, bw1_ref, fw0_ref, fw1_ref, xw1)

    @pl.when(l == 1)
    def _accum_l1():
        _accum(acc_ref, bw0_ref, bw1_ref, fw0_ref, fw1_ref,
               xw2_ref[pl.ds(k * BK, BK), :])

    @pl.when(k == NK - 1)
    def _epilogue():
        s = jnp.maximum(acc_ref[...] + bpre_ref[0], 0.0).astype(jnp.bfloat16)
        lin = (jnp.dot(s, wl_ref[0].astype(jnp.bfloat16),
                       preferred_element_type=jnp.float32) + blin_ref[0])

        @pl.when(l == 0)
        def _emit_l1():
            h1 = lin + embs_ref[...]
            h1_ref[pl.ds(i * BI, BI), :] = h1.astype(jnp.bfloat16)
            xw2_ref[pl.ds(i * BI, BI), :] = jnp.dot(
                h1.astype(jnp.bfloat16), wcat2_ref[...],
                preferred_element_type=jnp.float32).astype(jnp.bfloat16)

        @pl.when(l == 1)
        def _emit_out():
            out_ref[...] = lin + h1_ref[pl.ds(i * BI, BI), :].astype(jnp.float32)


def kernel(embs, fw_adj_0, fw_adj_1, bw_adj_0, bw_adj_1,
           W_fw, b_fw, W_bw, b_bw, W_lin, b_lin):
    Wcat = [jnp.concatenate(
        [W_bw[l, 0], W_bw[l, 1], W_fw[l, 0], W_fw[l, 1]], axis=1)
        for l in range(L)]
    bpre = jnp.stack([
        jnp.concatenate([b_bw[l, 0] + b_bw[l, 1], b_fw[l, 0] + b_fw[l, 1]])
        for l in range(L)])[:, None, :]
    blin = b_lin[:, None, :]
    wcat1_bf16 = Wcat[0].astype(jnp.bfloat16)
    wcat2_bf16 = Wcat[1].astype(jnp.bfloat16)

    adj_spec = pl.BlockSpec((BI, BK), lambda l, i, k: (i, k))
    out = pl.pallas_call(
        _mega_kernel,
        grid=(L, NI, NK),
        in_specs=[
            adj_spec, adj_spec, adj_spec, adj_spec,
            pl.BlockSpec((BK, D),
                         lambda l, i, k: (jnp.where(l == 0, k, 0), 0)),
            pl.BlockSpec((1, D, D), lambda l, i, k: (l, 0, 0)),
            pl.BlockSpec((1, 1, D), lambda l, i, k: (l, 0, 0)),
            pl.BlockSpec((1, 1, D), lambda l, i, k: (l, 0, 0)),
            pl.BlockSpec((BI, D),
                         lambda l, i, k: (jnp.where(l == 0, i, 0), 0)),
            pl.BlockSpec((D, 4 * H), lambda l, i, k: (0, 0)),
            pl.BlockSpec((D, 4 * H), lambda l, i, k: (0, 0)),
        ],
        out_specs=pl.BlockSpec((BI, D),
                               lambda l, i, k: (jnp.where(l == 1, i, 0), 0)),
        out_shape=jax.ShapeDtypeStruct((N, D), jnp.float32),
        scratch_shapes=[pltpu.VMEM((BI, D), jnp.float32),
                        pltpu.VMEM((N, D), jnp.bfloat16),
                        pltpu.VMEM((N, 4 * H), jnp.bfloat16)],
        compiler_params=pltpu.CompilerParams(
            dimension_semantics=("arbitrary", "arbitrary", "arbitrary"),
            vmem_limit_bytes=100 * 1024 * 1024),
    )(bw_adj_0, bw_adj_1, fw_adj_0, fw_adj_1, embs, W_lin,
      bpre, blin, embs, wcat1_bf16, wcat2_bf16)
    return out
